# baseline (speedup 1.0000x reference)
import jax
import jax.numpy as jnp
from jax.experimental import pallas as pl
from jax.experimental.pallas import tpu as pltpu

_N = 10000
_C = 256
_EPS = 1e-6


def _stage1_body(x_ref, wv_ref, wr_ref, val_ref, norm_ref, res_ref, s1_ref):
    x = x_ref[...]
    v = jnp.dot(x, wv_ref[...].T, preferred_element_type=jnp.float32)
    val_ref[...] = v
    nrm = jnp.sqrt(jnp.sum(v * v, axis=1, keepdims=True))
    norm_ref[...] = v / jnp.clip(nrm, 1e-12, None)
    res_ref[...] = jnp.dot(x, wr_ref[...].T, preferred_element_type=jnp.float32)
    s1_ref[...] = jnp.sum(jnp.abs(v), axis=1, keepdims=True)


def _stage2_body(out_ref, res_ref, wo_ref, o_ref):
    out = out_ref[...]
    wo = wo_ref[...]
    lin = jnp.dot(out, wo.T, preferred_element_type=jnp.float32)
    onrm = jnp.sqrt(jnp.sum(out * out, axis=1, keepdims=True))
    out_n = out / jnp.clip(onrm, 1e-12, None)
    wnrm = jnp.sqrt(jnp.sum(wo * wo, axis=1, keepdims=True))
    w_n = wo / jnp.clip(wnrm, 1e-12, None)
    cos2 = jnp.clip(jnp.dot(out_n, w_n.T, preferred_element_type=jnp.float32),
                    _EPS, 1.0)
    o_ref[...] = lin * cos2 + res_ref[...]


_BLK = 1000


def _stage1(x, Wv, Wr):
    grid = _N // _BLK
    return pl.pallas_call(
        _stage1_body,
        grid=(grid,),
        in_specs=[
            pl.BlockSpec((_BLK, _C), lambda i: (i, 0)),
            pl.BlockSpec((_C, _C), lambda i: (0, 0)),
            pl.BlockSpec((_C, _C), lambda i: (0, 0)),
        ],
        out_specs=[
            pl.BlockSpec((_BLK, _C), lambda i: (i, 0)),
            pl.BlockSpec((_BLK, _C), lambda i: (i, 0)),
            pl.BlockSpec((_BLK, _C), lambda i: (i, 0)),
            pl.BlockSpec((_BLK, 1), lambda i: (i, 0)),
        ],
        out_shape=[
            jax.ShapeDtypeStruct((_N, _C), jnp.float32),
            jax.ShapeDtypeStruct((_N, _C), jnp.float32),
            jax.ShapeDtypeStruct((_N, _C), jnp.float32),
            jax.ShapeDtypeStruct((_N, 1), jnp.float32),
        ],
    )(x, Wv, Wr)


def _stage2(out, res, Wo):
    grid = _N // _BLK
    return pl.pallas_call(
        _stage2_body,
        grid=(grid,),
        in_specs=[
            pl.BlockSpec((_BLK, _C), lambda i: (i, 0)),
            pl.BlockSpec((_BLK, _C), lambda i: (i, 0)),
            pl.BlockSpec((_C, _C), lambda i: (0, 0)),
        ],
        out_specs=pl.BlockSpec((_BLK, _C), lambda i: (i, 0)),
        out_shape=jax.ShapeDtypeStruct((_N, _C), jnp.float32),
    )(out, res, Wo)


def kernel(x, edge_index, Wv, Wo, Wr):
    row = edge_index[0].astype(jnp.int32)
    col = edge_index[1].astype(jnp.int32)
    val, norm, res, s1 = _stage1(x, Wv, Wr)

    cos = jnp.clip(jnp.sum(norm[col] * norm[row], axis=1), _EPS, 1.0)
    messages = val[col] * cos[:, None]
    out = jax.ops.segment_sum(messages, row, num_segments=_N)
    deg = jax.ops.segment_sum(jnp.ones((row.shape[0],), jnp.float32), row,
                              num_segments=_N)
    out = out / jnp.clip(deg, 1.0, None)[:, None]

    out_final = _stage2(out, res, Wo)

    contrib_mag = cos * s1[col, 0]
    sum_per_target = jax.ops.segment_sum(contrib_mag, row, num_segments=_N)
    contrib_norm = contrib_mag / jnp.clip(sum_per_target[row], 1e-12, None)
    return (out_final, jax.lax.stop_gradient(contrib_norm))


# speedup vs baseline: 3.8499x; 3.8499x over previous
import dataclasses
import functools

import jax
import jax.numpy as jnp
from jax import lax
from jax.experimental import pallas as pl
from jax.experimental.pallas import tpu as pltpu
from jax.experimental.pallas import tpu_sc as plsc

_N = 10000
_C = 256
_H = 128
_E = 160000
_EPS = 1e-6
_BE = 64
_NBLK = _E // _BE
_NW = 32
_NS = 16
_RPS = 624
_ZR = 104
_TAIL = _N - _RPS * _NS

_BLK = 1000


def _sc_compiler_params():
    cp = pltpu.CompilerParams()
    if "needs_layout_passes" in pltpu.CompilerParams.__dataclass_fields__:
        cp = dataclasses.replace(cp, needs_layout_passes=False)
    return cp


def _stage1_body(x_ref, wv_ref, wr_ref, val_ref, norm_ref, res_ref, s1_ref):
    x = x_ref[...]
    v = jnp.dot(x, wv_ref[...].T, preferred_element_type=jnp.float32)
    val_ref[...] = v
    nrm = jnp.sqrt(jnp.sum(v * v, axis=1, keepdims=True))
    norm_ref[...] = v / jnp.clip(nrm, 1e-12, None)
    res_ref[...] = jnp.dot(x, wr_ref[...].T, preferred_element_type=jnp.float32)
    s1_ref[...] = jnp.sum(jnp.abs(v), axis=1, keepdims=True)


def _stage1(x, Wv, Wr):
    return pl.pallas_call(
        _stage1_body,
        grid=(_N // _BLK,),
        in_specs=[
            pl.BlockSpec((_BLK, _C), lambda i: (i, 0)),
            pl.BlockSpec((_C, _C), lambda i: (0, 0)),
            pl.BlockSpec((_C, _C), lambda i: (0, 0)),
        ],
        out_specs=[
            pl.BlockSpec((_BLK, _C), lambda i: (i, 0)),
            pl.BlockSpec((_BLK, _C), lambda i: (i, 0)),
            pl.BlockSpec((_BLK, _C), lambda i: (i, 0)),
            pl.BlockSpec((_BLK, 1), lambda i: (i, 0)),
        ],
        out_shape=[
            jax.ShapeDtypeStruct((_N, _C), jnp.float32),
            jax.ShapeDtypeStruct((_N, _C), jnp.float32),
            jax.ShapeDtypeStruct((_N, _C), jnp.float32),
            jax.ShapeDtypeStruct((_N, 1), jnp.float32),
        ],
    )(x, Wv, Wr)


def _sc_cos_body(norm_hbm, row_hbm, col_hbm, s1_hbm,
                 cos_hbm, ct_hbm, spt_hbm, deg_hbm,
                 s1buf, sptacc, degacc, rbuf, cbuf,
                 rowidx, colidx, cosbuf, ctbuf, sem1, sem2):
    wid = lax.axis_index("s") * 2 + lax.axis_index("c")

    pltpu.sync_copy(s1_hbm, s1buf)

    @pl.loop(0, _N // 16)
    def _zero(i):
        z = jnp.zeros((16,), jnp.float32)
        sptacc[pl.ds(i * 16, 16)] = z
        degacc[pl.ds(i * 16, 16)] = z

    ones = jnp.ones((16,), jnp.float32)

    @pl.loop(wid, _NBLK, step=_NW)
    def _blk(b):
        base = b * _BE
        pltpu.sync_copy(row_hbm.at[pl.ds(base, _BE)], rowidx)
        pltpu.sync_copy(col_hbm.at[pl.ds(base, _BE)], colidx)
        cp1 = pltpu.async_copy(norm_hbm.at[rowidx], rbuf, sem1)
        cp2 = pltpu.async_copy(norm_hbm.at[colidx], cbuf, sem2)
        cp1.wait()
        cp2.wait()

        lanes = lax.iota(jnp.int32, 16)

        @pl.loop(0, _BE // 16)
        def _dotgrp(g):
            def edge_body(j, cosv):
                e = g * 16 + j
                acc = rbuf[e, pl.ds(0, 16)] * cbuf[e, pl.ds(0, 16)]
                for d in range(1, 16):
                    sl = pl.ds(d * 16, 16)
                    acc = acc + rbuf[e, sl] * cbuf[e, sl]
                cd = jnp.clip(jnp.sum(acc), _EPS, 1.0)
                return jnp.where(lanes == j, cd, cosv)

            cosv = lax.fori_loop(0, 16, edge_body,
                                 jnp.zeros((16,), jnp.float32))
            cosbuf[pl.ds(g * 16, 16)] = cosv

        @pl.loop(0, _BE // 16)
        def _grp(g):
            sl = pl.ds(g * 16, 16)
            civ = colidx[sl]
            riv = rowidx[sl]
            s1v = plsc.load_gather(s1buf, [civ])
            ctv = cosbuf[sl] * s1v
            ctbuf[sl] = ctv
            plsc.addupdate_scatter(sptacc, [riv], ctv)
            plsc.addupdate_scatter(degacc, [riv], ones)

        pltpu.sync_copy(cosbuf, cos_hbm.at[pl.ds(base, _BE)])
        pltpu.sync_copy(ctbuf, ct_hbm.at[pl.ds(base, _BE)])

    pltpu.sync_copy(sptacc, spt_hbm.at[wid])
    pltpu.sync_copy(degacc, deg_hbm.at[wid])


def _sc_cos(norm, row, col, s1):
    mesh = plsc.VectorSubcoreMesh(core_axis_name="c", subcore_axis_name="s")
    f32 = jnp.float32
    kern = pl.kernel(
        _sc_cos_body,
        mesh=mesh,
        out_type=[
            jax.ShapeDtypeStruct((_E,), f32),
            jax.ShapeDtypeStruct((_E,), f32),
            jax.ShapeDtypeStruct((_NW, _N), f32),
            jax.ShapeDtypeStruct((_NW, _N), f32),
        ],
        scratch_types=[
            pltpu.VMEM((_N,), f32),
            pltpu.VMEM((_N,), f32),
            pltpu.VMEM((_N,), f32),
            pltpu.VMEM((_BE, _C), f32),
            pltpu.VMEM((_BE, _C), f32),
            pltpu.VMEM((_BE,), jnp.int32),
            pltpu.VMEM((_BE,), jnp.int32),
            pltpu.VMEM((_BE,), f32),
            pltpu.VMEM((_BE,), f32),
            pltpu.SemaphoreType.DMA,
            pltpu.SemaphoreType.DMA,
        ],
        compiler_params=_sc_compiler_params(),
    )
    return kern(norm, row, col, s1)


def _ispt_body(spt_ref, deg_ref, o_ref, ideg_ref):
    s = jnp.sum(spt_ref[...], axis=0)
    o_ref[...] = (1.0 / jnp.clip(s, 1e-12, None))[None, :]
    d = jnp.sum(deg_ref[...], axis=0)
    ideg_ref[...] = (1.0 / jnp.clip(d, 1.0, None))[None, :]


def _ispt(spt_p, deg_p):
    return pl.pallas_call(
        _ispt_body,
        out_shape=[
            jax.ShapeDtypeStruct((1, _N), jnp.float32),
            jax.ShapeDtypeStruct((1, _N), jnp.float32),
        ],
    )(spt_p, deg_p)


def _sc_msg_body(val2_hbm, cos_hbm, ct_hbm, ispt_hbm, row_hbm, col_hbm,
                 out0_hbm, out1_hbm, cn_hbm,
                 acc_sh, zbuf, sptbuf, vbuf, rowidx, colidx, idxadj,
                 cosbuf, ctbuf, cnbuf, sem1):
    c = lax.axis_index("c")
    sid = lax.axis_index("s")

    @pl.loop(0, _ZR)
    def _z(i):
        for k in range(_H // 16):
            zbuf[i, pl.ds(k * 16, 16)] = jnp.zeros((16,), jnp.float32)

    @pl.loop(0, _RPS // _ZR)
    def _zs(k):
        pltpu.sync_copy(zbuf, acc_sh.at[pl.ds(sid * _RPS + k * _ZR, _ZR)])

    @pl.when(sid == _NS - 1)
    def _ztail():
        pltpu.sync_copy(zbuf.at[pl.ds(0, _TAIL)],
                        acc_sh.at[pl.ds(_RPS * _NS, _TAIL)])

    @pl.when(c == 0)
    def _():
        pltpu.sync_copy(ispt_hbm, sptbuf)

    plsc.subcore_barrier()

    coff = c * _N

    @pl.loop(sid, _NBLK, step=_NS)
    def _blk(b):
        base = b * _BE
        pltpu.sync_copy(row_hbm.at[pl.ds(base, _BE)], rowidx)
        pltpu.sync_copy(col_hbm.at[pl.ds(base, _BE)], colidx)

        @pl.loop(0, _BE // 16)
        def _adj(g):
            sl = pl.ds(g * 16, 16)
            idxadj[sl] = colidx[sl] + coff

        pltpu.async_copy(val2_hbm.at[idxadj], vbuf, sem1).wait()
        pltpu.sync_copy(cos_hbm.at[pl.ds(base, _BE)], cosbuf)

        lanes = lax.iota(jnp.int32, 16)

        @pl.loop(0, _BE // 16)
        def _sgrp(g):
            cosv = cosbuf[pl.ds(g * 16, 16)]

            def sbody(j, carry):
                sc = jnp.sum(jnp.where(lanes == j, cosv, 0.0))
                e = g * 16 + j
                for k in range(_H // 16):
                    sl = pl.ds(k * 16, 16)
                    vbuf[e, sl] = vbuf[e, sl] * sc
                return carry

            lax.fori_loop(0, 16, sbody, 0)

        pltpu.sync_copy(vbuf, acc_sh.at[rowidx], add=True)

        @pl.when(c == 0)
        def _():
            pltpu.sync_copy(ct_hbm.at[pl.ds(base, _BE)], ctbuf)

            @pl.loop(0, _BE // 16)
            def _cn(g):
                sl = pl.ds(g * 16, 16)
                iv = plsc.load_gather(sptbuf, [rowidx[sl]])
                cnbuf[sl] = ctbuf[sl] * iv

            pltpu.sync_copy(cnbuf, cn_hbm.at[pl.ds(base, _BE)])

    plsc.subcore_barrier()

    rows = pl.ds(sid * _RPS, _RPS)
    tail = pl.ds(_RPS * _NS, _TAIL)

    @pl.when(c == 0)
    def _():
        pltpu.sync_copy(acc_sh.at[rows], out0_hbm.at[rows])

        @pl.when(sid == _NS - 1)
        def _t0():
            pltpu.sync_copy(acc_sh.at[tail], out0_hbm.at[tail])

    @pl.when(c == 1)
    def _():
        pltpu.sync_copy(acc_sh.at[rows], out1_hbm.at[rows])

        @pl.when(sid == _NS - 1)
        def _t1():
            pltpu.sync_copy(acc_sh.at[tail], out1_hbm.at[tail])


def _sc_msg(val2, cos, ct, ispt, row, col):
    mesh = plsc.VectorSubcoreMesh(core_axis_name="c", subcore_axis_name="s")
    f32 = jnp.float32
    kern = pl.kernel(
        _sc_msg_body,
        mesh=mesh,
        out_type=[
            jax.ShapeDtypeStruct((_N, _H), f32),
            jax.ShapeDtypeStruct((_N, _H), f32),
            jax.ShapeDtypeStruct((_E,), f32),
        ],
        scratch_types=[
            pltpu.VMEM_SHARED((_N, _H), f32),
            pltpu.VMEM((_ZR, _H), f32),
            pltpu.VMEM((_N,), f32),
            pltpu.VMEM((_BE, _H), f32),
            pltpu.VMEM((_BE,), jnp.int32),
            pltpu.VMEM((_BE,), jnp.int32),
            pltpu.VMEM((_BE,), jnp.int32),
            pltpu.VMEM((_BE,), f32),
            pltpu.VMEM((_BE,), f32),
            pltpu.VMEM((_BE,), f32),
            pltpu.SemaphoreType.DMA,
        ],
        compiler_params=_sc_compiler_params(),
    )
    return kern(val2, cos, ct, ispt, row, col)


def _stage2_body(out0_ref, out1_ref, ideg_ref, res_ref, wo_ref, o_ref):
    out = jnp.concatenate([out0_ref[...], out1_ref[...]], axis=1)
    out = out * ideg_ref[...]
    wo = wo_ref[...]
    lin = jnp.dot(out, wo.T, preferred_element_type=jnp.float32)
    onrm = jnp.sqrt(jnp.sum(out * out, axis=1, keepdims=True))
    out_n = out / jnp.clip(onrm, 1e-12, None)
    wnrm = jnp.sqrt(jnp.sum(wo * wo, axis=1, keepdims=True))
    w_n = wo / jnp.clip(wnrm, 1e-12, None)
    cos2 = jnp.clip(jnp.dot(out_n, w_n.T, preferred_element_type=jnp.float32),
                    _EPS, 1.0)
    o_ref[...] = lin * cos2 + res_ref[...]


def _stage2(out0, out1, ideg, res, Wo):
    return pl.pallas_call(
        _stage2_body,
        grid=(_N // _BLK,),
        in_specs=[
            pl.BlockSpec((_BLK, _H), lambda i: (i, 0)),
            pl.BlockSpec((_BLK, _H), lambda i: (i, 0)),
            pl.BlockSpec((_BLK, 1), lambda i: (i, 0)),
            pl.BlockSpec((_BLK, _C), lambda i: (i, 0)),
            pl.BlockSpec((_C, _C), lambda i: (0, 0)),
        ],
        out_specs=pl.BlockSpec((_BLK, _C), lambda i: (i, 0)),
        out_shape=jax.ShapeDtypeStruct((_N, _C), jnp.float32),
    )(out0, out1, ideg, res, Wo)


def kernel(x, edge_index, Wv, Wo, Wr):
    row = edge_index[0].astype(jnp.int32)
    col = edge_index[1].astype(jnp.int32)
    val, norm, res, s1 = _stage1(x, Wv, Wr)
    s1 = s1.reshape(_N)

    cos, ct, spt_p, deg_p = _sc_cos(norm, row, col, s1)
    ispt2, ideg = _ispt(spt_p, deg_p)
    ispt = ispt2.reshape(_N)

    val2 = jnp.concatenate([val[:, :_H], val[:, _H:]], axis=0)
    out0, out1, contrib_norm = _sc_msg(val2, cos, ct, ispt, row, col)

    out_final = _stage2(out0, out1, ideg.reshape(_N, 1), res, Wo)
    return (out_final, jax.lax.stop_gradient(contrib_norm))


# speedup vs baseline: 4.4802x; 1.1637x over previous
import dataclasses
import functools

import jax
import jax.numpy as jnp
from jax import lax
from jax.experimental import pallas as pl
from jax.experimental.pallas import tpu as pltpu
from jax.experimental.pallas import tpu_sc as plsc

_N = 10000
_C = 256
_H = 128
_E = 160000
_EPS = 1e-6
_BE = 64
_NBLK = _E // _BE
_NW = 32
_NS = 16
_RPS = 624
_ZR = 104
_TAIL = _N - _RPS * _NS

_BLK = 1000


def _sc_compiler_params():
    cp = pltpu.CompilerParams()
    if "needs_layout_passes" in pltpu.CompilerParams.__dataclass_fields__:
        cp = dataclasses.replace(cp, needs_layout_passes=False)
    return cp


def _stage1_body(x_ref, wv_ref, wr_ref, val_ref, norm_ref, res_ref, s1_ref):
    x = x_ref[...]
    v = jnp.dot(x, wv_ref[...].T, preferred_element_type=jnp.float32)
    val_ref[...] = v
    nrm = jnp.sqrt(jnp.sum(v * v, axis=1, keepdims=True))
    norm_ref[...] = v / jnp.clip(nrm, 1e-12, None)
    res_ref[...] = jnp.dot(x, wr_ref[...].T, preferred_element_type=jnp.float32)
    s1_ref[...] = jnp.sum(jnp.abs(v), axis=1, keepdims=True)


def _stage1(x, Wv, Wr):
    return pl.pallas_call(
        _stage1_body,
        grid=(_N // _BLK,),
        in_specs=[
            pl.BlockSpec((_BLK, _C), lambda i: (i, 0)),
            pl.BlockSpec((_C, _C), lambda i: (0, 0)),
            pl.BlockSpec((_C, _C), lambda i: (0, 0)),
        ],
        out_specs=[
            pl.BlockSpec((_BLK, _C), lambda i: (i, 0)),
            pl.BlockSpec((_BLK, _C), lambda i: (i, 0)),
            pl.BlockSpec((_BLK, _C), lambda i: (i, 0)),
            pl.BlockSpec((_BLK, 1), lambda i: (i, 0)),
        ],
        out_shape=[
            jax.ShapeDtypeStruct((_N, _C), jnp.float32),
            jax.ShapeDtypeStruct((_N, _C), jnp.float32),
            jax.ShapeDtypeStruct((_N, _C), jnp.float32),
            jax.ShapeDtypeStruct((_N, 1), jnp.float32),
        ],
    )(x, Wv, Wr)


_PAIRS = _NBLK // (2 * _NW)
_TAILB = _NBLK - 2 * _NW * _PAIRS


def _sc_cos_body(norm_hbm, row_hbm, col_hbm, s1_hbm,
                 cos_hbm, ct_hbm, spt_hbm, deg_hbm,
                 s1buf, sptacc, degacc,
                 rbuf0, cbuf0, rbuf1, cbuf1,
                 rowidx0, colidx0, rowidx1, colidx1,
                 cosbuf, ctbuf, semr0, semc0, semr1, semc1):
    wid = lax.axis_index("s") * 2 + lax.axis_index("c")

    pltpu.sync_copy(s1_hbm, s1buf)

    @pl.loop(0, _N // 16)
    def _zero(i):
        z = jnp.zeros((16,), jnp.float32)
        sptacc[pl.ds(i * 16, 16)] = z
        degacc[pl.ds(i * 16, 16)] = z

    ones = jnp.ones((16,), jnp.float32)
    lanes = lax.iota(jnp.int32, 16)

    def issue(b, rowidx, colidx, rbuf, cbuf, semr, semc):
        base = b * _BE
        pltpu.sync_copy(row_hbm.at[pl.ds(base, _BE)], rowidx)
        pltpu.sync_copy(col_hbm.at[pl.ds(base, _BE)], colidx)
        pltpu.async_copy(norm_hbm.at[rowidx], rbuf, semr)
        pltpu.async_copy(norm_hbm.at[colidx], cbuf, semc)

    def wait(rowidx, colidx, rbuf, cbuf, semr, semc):
        pltpu.make_async_copy(norm_hbm.at[rowidx], rbuf, semr).wait()
        pltpu.make_async_copy(norm_hbm.at[colidx], cbuf, semc).wait()

    def compute(b, rowidx, colidx, rbuf, cbuf):
        base = b * _BE

        @pl.loop(0, _BE // 16)
        def _dotgrp(g):
            def edge_body(j, cosv):
                e = g * 16 + j
                acc = rbuf[e, pl.ds(0, 16)] * cbuf[e, pl.ds(0, 16)]
                for d in range(1, 16):
                    sl = pl.ds(d * 16, 16)
                    acc = acc + rbuf[e, sl] * cbuf[e, sl]
                cd = jnp.clip(jnp.sum(acc), _EPS, 1.0)
                return jnp.where(lanes == j, cd, cosv)

            cosv = lax.fori_loop(0, 16, edge_body,
                                 jnp.zeros((16,), jnp.float32))
            cosbuf[pl.ds(g * 16, 16)] = cosv

        @pl.loop(0, _BE // 16)
        def _grp(g):
            sl = pl.ds(g * 16, 16)
            civ = colidx[sl]
            riv = rowidx[sl]
            s1v = plsc.load_gather(s1buf, [civ])
            ctv = cosbuf[sl] * s1v
            ctbuf[sl] = ctv
            plsc.addupdate_scatter(sptacc, [riv], ctv)
            plsc.addupdate_scatter(degacc, [riv], ones)

        pltpu.sync_copy(cosbuf, cos_hbm.at[pl.ds(base, _BE)])
        pltpu.sync_copy(ctbuf, ct_hbm.at[pl.ds(base, _BE)])

    bufs0 = (rowidx0, colidx0, rbuf0, cbuf0, semr0, semc0)
    bufs1 = (rowidx1, colidx1, rbuf1, cbuf1, semr1, semc1)

    issue(wid, *bufs0)

    @pl.loop(0, _PAIRS)
    def _pair(k):
        b1 = wid + k * (2 * _NW)
        b2 = b1 + _NW
        issue(b2, *bufs1)
        wait(*bufs0)
        compute(b1, rowidx0, colidx0, rbuf0, cbuf0)

        @pl.when(k < _PAIRS - 1)
        def _():
            issue(b1 + 2 * _NW, *bufs0)

        wait(*bufs1)
        compute(b2, rowidx1, colidx1, rbuf1, cbuf1)

    @pl.when(wid < _TAILB)
    def _tail():
        b = 2 * _NW * _PAIRS + wid
        issue(b, *bufs0)
        wait(*bufs0)
        compute(b, rowidx0, colidx0, rbuf0, cbuf0)

    pltpu.sync_copy(sptacc, spt_hbm.at[wid])
    pltpu.sync_copy(degacc, deg_hbm.at[wid])


def _sc_cos(norm, row, col, s1):
    mesh = plsc.VectorSubcoreMesh(core_axis_name="c", subcore_axis_name="s")
    f32 = jnp.float32
    kern = pl.kernel(
        _sc_cos_body,
        mesh=mesh,
        out_type=[
            jax.ShapeDtypeStruct((_E,), f32),
            jax.ShapeDtypeStruct((_E,), f32),
            jax.ShapeDtypeStruct((_NW, _N), f32),
            jax.ShapeDtypeStruct((_NW, _N), f32),
        ],
        scratch_types=[
            pltpu.VMEM((_N,), f32),
            pltpu.VMEM((_N,), f32),
            pltpu.VMEM((_N,), f32),
            pltpu.VMEM((_BE, _C), f32),
            pltpu.VMEM((_BE, _C), f32),
            pltpu.VMEM((_BE, _C), f32),
            pltpu.VMEM((_BE, _C), f32),
            pltpu.VMEM((_BE,), jnp.int32),
            pltpu.VMEM((_BE,), jnp.int32),
            pltpu.VMEM((_BE,), jnp.int32),
            pltpu.VMEM((_BE,), jnp.int32),
            pltpu.VMEM((_BE,), f32),
            pltpu.VMEM((_BE,), f32),
            pltpu.SemaphoreType.DMA,
            pltpu.SemaphoreType.DMA,
            pltpu.SemaphoreType.DMA,
            pltpu.SemaphoreType.DMA,
        ],
        compiler_params=_sc_compiler_params(),
    )
    return kern(norm, row, col, s1)


def _ispt_body(spt_ref, deg_ref, o_ref, ideg_ref):
    s = jnp.sum(spt_ref[...], axis=0)
    o_ref[...] = (1.0 / jnp.clip(s, 1e-12, None))[None, :]
    d = jnp.sum(deg_ref[...], axis=0)
    ideg_ref[...] = (1.0 / jnp.clip(d, 1.0, None))[None, :]


def _ispt(spt_p, deg_p):
    return pl.pallas_call(
        _ispt_body,
        out_shape=[
            jax.ShapeDtypeStruct((1, _N), jnp.float32),
            jax.ShapeDtypeStruct((1, _N), jnp.float32),
        ],
    )(spt_p, deg_p)


def _sc_msg_body(val2_hbm, cos_hbm, ct_hbm, ispt_hbm, row_hbm, col_hbm,
                 out0_hbm, out1_hbm, cn_hbm,
                 acc_sh, zbuf, sptbuf, vbuf, rowidx, colidx, idxadj,
                 cosbuf, ctbuf, cnbuf, sem1):
    c = lax.axis_index("c")
    sid = lax.axis_index("s")

    @pl.loop(0, _ZR)
    def _z(i):
        for k in range(_H // 16):
            zbuf[i, pl.ds(k * 16, 16)] = jnp.zeros((16,), jnp.float32)

    @pl.loop(0, _RPS // _ZR)
    def _zs(k):
        pltpu.sync_copy(zbuf, acc_sh.at[pl.ds(sid * _RPS + k * _ZR, _ZR)])

    @pl.when(sid == _NS - 1)
    def _ztail():
        pltpu.sync_copy(zbuf.at[pl.ds(0, _TAIL)],
                        acc_sh.at[pl.ds(_RPS * _NS, _TAIL)])

    pltpu.sync_copy(ispt_hbm, sptbuf)

    plsc.subcore_barrier()

    coff = c * _N

    @pl.loop(sid, _NBLK, step=_NS)
    def _blk(b):
        base = b * _BE
        pltpu.sync_copy(row_hbm.at[pl.ds(base, _BE)], rowidx)
        pltpu.sync_copy(col_hbm.at[pl.ds(base, _BE)], colidx)

        @pl.loop(0, _BE // 16)
        def _adj(g):
            sl = pl.ds(g * 16, 16)
            idxadj[sl] = colidx[sl] + coff

        pltpu.async_copy(val2_hbm.at[idxadj], vbuf, sem1).wait()
        pltpu.sync_copy(cos_hbm.at[pl.ds(base, _BE)], cosbuf)

        lanes = lax.iota(jnp.int32, 16)

        @pl.loop(0, _BE // 16)
        def _sgrp(g):
            cosv = cosbuf[pl.ds(g * 16, 16)]

            def sbody(j, carry):
                sc = jnp.sum(jnp.where(lanes == j, cosv, 0.0))
                e = g * 16 + j
                for k in range(_H // 16):
                    sl = pl.ds(k * 16, 16)
                    vbuf[e, sl] = vbuf[e, sl] * sc
                return carry

            lax.fori_loop(0, 16, sbody, 0)

        pltpu.sync_copy(vbuf, acc_sh.at[rowidx], add=True)

        @pl.when(lax.rem(b, 2) == c)
        def _():
            pltpu.sync_copy(ct_hbm.at[pl.ds(base, _BE)], ctbuf)

            @pl.loop(0, _BE // 16)
            def _cn(g):
                sl = pl.ds(g * 16, 16)
                iv = plsc.load_gather(sptbuf, [rowidx[sl]])
                cnbuf[sl] = ctbuf[sl] * iv

            pltpu.sync_copy(cnbuf, cn_hbm.at[pl.ds(base, _BE)])

    plsc.subcore_barrier()

    rows = pl.ds(sid * _RPS, _RPS)
    tail = pl.ds(_RPS * _NS, _TAIL)

    @pl.when(c == 0)
    def _():
        pltpu.sync_copy(acc_sh.at[rows], out0_hbm.at[rows])

        @pl.when(sid == _NS - 1)
        def _t0():
            pltpu.sync_copy(acc_sh.at[tail], out0_hbm.at[tail])

    @pl.when(c == 1)
    def _():
        pltpu.sync_copy(acc_sh.at[rows], out1_hbm.at[rows])

        @pl.when(sid == _NS - 1)
        def _t1():
            pltpu.sync_copy(acc_sh.at[tail], out1_hbm.at[tail])


def _sc_msg(val2, cos, ct, ispt, row, col):
    mesh = plsc.VectorSubcoreMesh(core_axis_name="c", subcore_axis_name="s")
    f32 = jnp.float32
    kern = pl.kernel(
        _sc_msg_body,
        mesh=mesh,
        out_type=[
            jax.ShapeDtypeStruct((_N, _H), f32),
            jax.ShapeDtypeStruct((_N, _H), f32),
            jax.ShapeDtypeStruct((_E,), f32),
        ],
        scratch_types=[
            pltpu.VMEM_SHARED((_N, _H), f32),
            pltpu.VMEM((_ZR, _H), f32),
            pltpu.VMEM((_N,), f32),
            pltpu.VMEM((_BE, _H), f32),
            pltpu.VMEM((_BE,), jnp.int32),
            pltpu.VMEM((_BE,), jnp.int32),
            pltpu.VMEM((_BE,), jnp.int32),
            pltpu.VMEM((_BE,), f32),
            pltpu.VMEM((_BE,), f32),
            pltpu.VMEM((_BE,), f32),
            pltpu.SemaphoreType.DMA,
        ],
        compiler_params=_sc_compiler_params(),
    )
    return kern(val2, cos, ct, ispt, row, col)


def _stage2_body(out0_ref, out1_ref, ideg_ref, res_ref, wo_ref, o_ref):
    out = jnp.concatenate([out0_ref[...], out1_ref[...]], axis=1)
    out = out * ideg_ref[...]
    wo = wo_ref[...]
    lin = jnp.dot(out, wo.T, preferred_element_type=jnp.float32)
    onrm = jnp.sqrt(jnp.sum(out * out, axis=1, keepdims=True))
    out_n = out / jnp.clip(onrm, 1e-12, None)
    wnrm = jnp.sqrt(jnp.sum(wo * wo, axis=1, keepdims=True))
    w_n = wo / jnp.clip(wnrm, 1e-12, None)
    cos2 = jnp.clip(jnp.dot(out_n, w_n.T, preferred_element_type=jnp.float32),
                    _EPS, 1.0)
    o_ref[...] = lin * cos2 + res_ref[...]


def _stage2(out0, out1, ideg, res, Wo):
    return pl.pallas_call(
        _stage2_body,
        grid=(_N // _BLK,),
        in_specs=[
            pl.BlockSpec((_BLK, _H), lambda i: (i, 0)),
            pl.BlockSpec((_BLK, _H), lambda i: (i, 0)),
            pl.BlockSpec((_BLK, 1), lambda i: (i, 0)),
            pl.BlockSpec((_BLK, _C), lambda i: (i, 0)),
            pl.BlockSpec((_C, _C), lambda i: (0, 0)),
        ],
        out_specs=pl.BlockSpec((_BLK, _C), lambda i: (i, 0)),
        out_shape=jax.ShapeDtypeStruct((_N, _C), jnp.float32),
    )(out0, out1, ideg, res, Wo)


def kernel(x, edge_index, Wv, Wo, Wr):
    row = edge_index[0].astype(jnp.int32)
    col = edge_index[1].astype(jnp.int32)
    val, norm, res, s1 = _stage1(x, Wv, Wr)
    s1 = s1.reshape(_N)

    cos, ct, spt_p, deg_p = _sc_cos(norm, row, col, s1)
    ispt2, ideg = _ispt(spt_p, deg_p)
    ispt = ispt2.reshape(_N)

    val2 = jnp.concatenate([val[:, :_H], val[:, _H:]], axis=0)
    out0, out1, contrib_norm = _sc_msg(val2, cos, ct, ispt, row, col)

    out_final = _stage2(out0, out1, ideg.reshape(_N, 1), res, Wo)
    return (out_final, jax.lax.stop_gradient(contrib_norm))


# speedup vs baseline: 5.9059x; 1.3182x over previous
import dataclasses
import functools

import jax
import jax.numpy as jnp
from jax import lax
from jax.experimental import pallas as pl
from jax.experimental.pallas import tpu as pltpu
from jax.experimental.pallas import tpu_sc as plsc

_N = 10000
_C = 256
_H = 128
_E = 160000
_EPS = 1e-6
_BE = 64
_NBLK = _E // _BE
_NW = 32
_NS = 16
_RPS = 624
_ZR = 104
_TAIL = _N - _RPS * _NS

_BLK = 1000


def _sc_compiler_params():
    cp = pltpu.CompilerParams()
    if "needs_layout_passes" in pltpu.CompilerParams.__dataclass_fields__:
        cp = dataclasses.replace(cp, needs_layout_passes=False)
    return cp


def _stage1_body(x_ref, wv_ref, wr_ref, val_ref, norm_ref, res_ref, s1_ref):
    x = x_ref[...]
    v = jnp.dot(x, wv_ref[...].T, preferred_element_type=jnp.float32)
    val_ref[...] = v
    nrm = jnp.sqrt(jnp.sum(v * v, axis=1, keepdims=True))
    norm_ref[...] = v / jnp.clip(nrm, 1e-12, None)
    res_ref[...] = jnp.dot(x, wr_ref[...].T, preferred_element_type=jnp.float32)
    s1_ref[...] = jnp.sum(jnp.abs(v), axis=1, keepdims=True)


def _stage1(x, Wv, Wr):
    return pl.pallas_call(
        _stage1_body,
        grid=(_N // _BLK,),
        in_specs=[
            pl.BlockSpec((_BLK, _C), lambda i: (i, 0)),
            pl.BlockSpec((_C, _C), lambda i: (0, 0)),
            pl.BlockSpec((_C, _C), lambda i: (0, 0)),
        ],
        out_specs=[
            pl.BlockSpec((_BLK, _C), lambda i: (i, 0)),
            pl.BlockSpec((_BLK, _C), lambda i: (i, 0)),
            pl.BlockSpec((_BLK, _C), lambda i: (i, 0)),
            pl.BlockSpec((_BLK, 1), lambda i: (i, 0)),
        ],
        out_shape=[
            jax.ShapeDtypeStruct((_N, _C), jnp.float32),
            jax.ShapeDtypeStruct((_N, _C), jnp.float32),
            jax.ShapeDtypeStruct((_N, _C), jnp.float32),
            jax.ShapeDtypeStruct((_N, 1), jnp.float32),
        ],
    )(x, Wv, Wr)


_PAIRS = _NBLK // (2 * _NW)
_TAILB = _NBLK - 2 * _NW * _PAIRS


def _sc_cos_body(norm_hbm, row_hbm, col_hbm, s1_hbm,
                 cos_hbm, ct_hbm, spt_hbm, deg_hbm,
                 s1buf, sptacc, degacc,
                 rbuf0, cbuf0, rbuf1, cbuf1,
                 rowidx0, colidx0, rowidx1, colidx1,
                 cosbuf, ctbuf, semr0, semc0, semr1, semc1):
    wid = lax.axis_index("s") * 2 + lax.axis_index("c")

    pltpu.sync_copy(s1_hbm, s1buf)

    @pl.loop(0, _N // 16)
    def _zero(i):
        z = jnp.zeros((16,), jnp.float32)
        sptacc[pl.ds(i * 16, 16)] = z
        degacc[pl.ds(i * 16, 16)] = z

    ones = jnp.ones((16,), jnp.float32)
    lanes = lax.iota(jnp.int32, 16)

    def issue(b, rowidx, colidx, rbuf, cbuf, semr, semc):
        base = b * _BE
        pltpu.sync_copy(row_hbm.at[pl.ds(base, _BE)], rowidx)
        pltpu.sync_copy(col_hbm.at[pl.ds(base, _BE)], colidx)
        pltpu.async_copy(norm_hbm.at[rowidx], rbuf, semr)
        pltpu.async_copy(norm_hbm.at[colidx], cbuf, semc)

    def wait(rowidx, colidx, rbuf, cbuf, semr, semc):
        pltpu.make_async_copy(norm_hbm.at[rowidx], rbuf, semr).wait()
        pltpu.make_async_copy(norm_hbm.at[colidx], cbuf, semc).wait()

    def compute(b, rowidx, colidx, rbuf, cbuf):
        base = b * _BE

        @pl.loop(0, _BE // 16)
        def _dotgrp(g):
            def edge_body(j, cosv):
                e = g * 16 + j
                acc = rbuf[e, pl.ds(0, 16)] * cbuf[e, pl.ds(0, 16)]
                for d in range(1, 16):
                    sl = pl.ds(d * 16, 16)
                    acc = acc + rbuf[e, sl] * cbuf[e, sl]
                cd = jnp.clip(jnp.sum(acc), _EPS, 1.0)
                return jnp.where(lanes == j, cd, cosv)

            cosv = lax.fori_loop(0, 16, edge_body,
                                 jnp.zeros((16,), jnp.float32))
            cosbuf[pl.ds(g * 16, 16)] = cosv

        @pl.loop(0, _BE // 16)
        def _grp(g):
            sl = pl.ds(g * 16, 16)
            civ = colidx[sl]
            riv = rowidx[sl]
            s1v = plsc.load_gather(s1buf, [civ])
            ctv = cosbuf[sl] * s1v
            ctbuf[sl] = ctv
            plsc.addupdate_scatter(sptacc, [riv], ctv)
            plsc.addupdate_scatter(degacc, [riv], ones)

        pltpu.sync_copy(cosbuf, cos_hbm.at[pl.ds(base, _BE)])
        pltpu.sync_copy(ctbuf, ct_hbm.at[pl.ds(base, _BE)])

    bufs0 = (rowidx0, colidx0, rbuf0, cbuf0, semr0, semc0)
    bufs1 = (rowidx1, colidx1, rbuf1, cbuf1, semr1, semc1)

    issue(wid, *bufs0)

    @pl.loop(0, _PAIRS)
    def _pair(k):
        b1 = wid + k * (2 * _NW)
        b2 = b1 + _NW
        issue(b2, *bufs1)
        wait(*bufs0)
        compute(b1, rowidx0, colidx0, rbuf0, cbuf0)

        @pl.when(k < _PAIRS - 1)
        def _():
            issue(b1 + 2 * _NW, *bufs0)

        wait(*bufs1)
        compute(b2, rowidx1, colidx1, rbuf1, cbuf1)

    @pl.when(wid < _TAILB)
    def _tail():
        b = 2 * _NW * _PAIRS + wid
        issue(b, *bufs0)
        wait(*bufs0)
        compute(b, rowidx0, colidx0, rbuf0, cbuf0)

    pltpu.sync_copy(sptacc, spt_hbm.at[wid])
    pltpu.sync_copy(degacc, deg_hbm.at[wid])


def _sc_cos(norm, row, col, s1):
    mesh = plsc.VectorSubcoreMesh(core_axis_name="c", subcore_axis_name="s")
    f32 = jnp.float32
    kern = pl.kernel(
        _sc_cos_body,
        mesh=mesh,
        out_type=[
            jax.ShapeDtypeStruct((_E,), f32),
            jax.ShapeDtypeStruct((_E,), f32),
            jax.ShapeDtypeStruct((_NW, _N), f32),
            jax.ShapeDtypeStruct((_NW, _N), f32),
        ],
        scratch_types=[
            pltpu.VMEM((_N,), f32),
            pltpu.VMEM((_N,), f32),
            pltpu.VMEM((_N,), f32),
            pltpu.VMEM((_BE, _C), f32),
            pltpu.VMEM((_BE, _C), f32),
            pltpu.VMEM((_BE, _C), f32),
            pltpu.VMEM((_BE, _C), f32),
            pltpu.VMEM((_BE,), jnp.int32),
            pltpu.VMEM((_BE,), jnp.int32),
            pltpu.VMEM((_BE,), jnp.int32),
            pltpu.VMEM((_BE,), jnp.int32),
            pltpu.VMEM((_BE,), f32),
            pltpu.VMEM((_BE,), f32),
            pltpu.SemaphoreType.DMA,
            pltpu.SemaphoreType.DMA,
            pltpu.SemaphoreType.DMA,
            pltpu.SemaphoreType.DMA,
        ],
        compiler_params=_sc_compiler_params(),
    )
    return kern(norm, row, col, s1)


def _ispt_body(spt_ref, deg_ref, o_ref, ideg_ref):
    s = jnp.sum(spt_ref[...], axis=0)
    o_ref[...] = (1.0 / jnp.clip(s, 1e-12, None))[None, :]
    d = jnp.sum(deg_ref[...], axis=0)
    ideg_ref[...] = (1.0 / jnp.clip(d, 1.0, None))[None, :]


def _ispt(spt_p, deg_p):
    return pl.pallas_call(
        _ispt_body,
        out_shape=[
            jax.ShapeDtypeStruct((1, _N), jnp.float32),
            jax.ShapeDtypeStruct((1, _N), jnp.float32),
        ],
    )(spt_p, deg_p)


_PAIRS2 = _NBLK // (2 * _NS)
_TAILB2 = _NBLK - 2 * _NS * _PAIRS2


def _sc_msg_body(val2_hbm, cos_hbm, ct_hbm, ispt_hbm, row_hbm, col_hbm,
                 out0_hbm, out1_hbm, cn_hbm,
                 acc_sh, zbuf, sptbuf,
                 vbuf0, rowidx0, colidx0, idxadj0,
                 vbuf1, rowidx1, colidx1, idxadj1,
                 cosbuf, ctbuf, cnbuf,
                 semg0, semg1, sems0, sems1):
    c = lax.axis_index("c")
    sid = lax.axis_index("s")

    @pl.loop(0, _ZR)
    def _z(i):
        for k in range(_H // 16):
            zbuf[i, pl.ds(k * 16, 16)] = jnp.zeros((16,), jnp.float32)

    @pl.loop(0, _RPS // _ZR)
    def _zs(k):
        pltpu.sync_copy(zbuf, acc_sh.at[pl.ds(sid * _RPS + k * _ZR, _ZR)])

    @pl.when(sid == _NS - 1)
    def _ztail():
        pltpu.sync_copy(zbuf.at[pl.ds(0, _TAIL)],
                        acc_sh.at[pl.ds(_RPS * _NS, _TAIL)])

    pltpu.sync_copy(ispt_hbm, sptbuf)

    plsc.subcore_barrier()

    coff = c * _N
    lanes = lax.iota(jnp.int32, 16)

    def issue_gather(b, rowidx, colidx, idxadj, vbuf, semg):
        base = b * _BE
        pltpu.sync_copy(row_hbm.at[pl.ds(base, _BE)], rowidx)
        pltpu.sync_copy(col_hbm.at[pl.ds(base, _BE)], colidx)

        @pl.loop(0, _BE // 16)
        def _adj(g):
            sl = pl.ds(g * 16, 16)
            idxadj[sl] = colidx[sl] + coff

        pltpu.async_copy(val2_hbm.at[idxadj], vbuf, semg)

    def wait_gather(idxadj, vbuf, semg):
        pltpu.make_async_copy(val2_hbm.at[idxadj], vbuf, semg).wait()

    def wait_scatter(rowidx, vbuf, sems):
        pltpu.make_async_copy(vbuf, acc_sh.at[rowidx], sems).wait()

    def compute(b, rowidx, vbuf, sems):
        base = b * _BE
        pltpu.sync_copy(cos_hbm.at[pl.ds(base, _BE)], cosbuf)

        @pl.loop(0, _BE // 16)
        def _sgrp(g):
            cosv = cosbuf[pl.ds(g * 16, 16)]

            def sbody(j, carry):
                sc = jnp.sum(jnp.where(lanes == j, cosv, 0.0))
                e = g * 16 + j
                for k in range(_H // 16):
                    sl = pl.ds(k * 16, 16)
                    vbuf[e, sl] = vbuf[e, sl] * sc
                return carry

            lax.fori_loop(0, 16, sbody, 0)

        pltpu.async_copy(vbuf, acc_sh.at[rowidx], sems, add=True)

        @pl.when(lax.rem(b, 2) == c)
        def _():
            pltpu.sync_copy(ct_hbm.at[pl.ds(base, _BE)], ctbuf)

            @pl.loop(0, _BE // 16)
            def _cn(g):
                sl = pl.ds(g * 16, 16)
                iv = plsc.load_gather(sptbuf, [rowidx[sl]])
                cnbuf[sl] = ctbuf[sl] * iv

            pltpu.sync_copy(cnbuf, cn_hbm.at[pl.ds(base, _BE)])

    set0 = (rowidx0, colidx0, idxadj0, vbuf0, semg0, sems0)
    set1 = (rowidx1, colidx1, idxadj1, vbuf1, semg1, sems1)

    def issue_set(b, s):
        rowidx, colidx, idxadj, vbuf, semg, sems = s
        issue_gather(b, rowidx, colidx, idxadj, vbuf, semg)

    issue_set(sid, set0)

    @pl.loop(0, _PAIRS2)
    def _pair(k):
        b1 = sid + k * (2 * _NS)
        b2 = b1 + _NS

        @pl.when(k > 0)
        def _():
            wait_scatter(rowidx1, vbuf1, sems1)

        issue_set(b2, set1)
        wait_gather(idxadj0, vbuf0, semg0)
        compute(b1, rowidx0, vbuf0, sems0)

        @pl.when(k < _PAIRS2 - 1)
        def _():
            wait_scatter(rowidx0, vbuf0, sems0)
            issue_set(b1 + 2 * _NS, set0)

        wait_gather(idxadj1, vbuf1, semg1)
        compute(b2, rowidx1, vbuf1, sems1)

    wait_scatter(rowidx0, vbuf0, sems0)
    wait_scatter(rowidx1, vbuf1, sems1)

    @pl.when(sid < _TAILB2)
    def _tail():
        b = 2 * _NS * _PAIRS2 + sid
        issue_set(b, set0)
        wait_gather(idxadj0, vbuf0, semg0)
        compute(b, rowidx0, vbuf0, sems0)
        wait_scatter(rowidx0, vbuf0, sems0)

    plsc.subcore_barrier()

    rows = pl.ds(sid * _RPS, _RPS)
    tail = pl.ds(_RPS * _NS, _TAIL)

    @pl.when(c == 0)
    def _():
        pltpu.sync_copy(acc_sh.at[rows], out0_hbm.at[rows])

        @pl.when(sid == _NS - 1)
        def _t0():
            pltpu.sync_copy(acc_sh.at[tail], out0_hbm.at[tail])

    @pl.when(c == 1)
    def _():
        pltpu.sync_copy(acc_sh.at[rows], out1_hbm.at[rows])

        @pl.when(sid == _NS - 1)
        def _t1():
            pltpu.sync_copy(acc_sh.at[tail], out1_hbm.at[tail])


def _sc_msg(val2, cos, ct, ispt, row, col):
    mesh = plsc.VectorSubcoreMesh(core_axis_name="c", subcore_axis_name="s")
    f32 = jnp.float32
    kern = pl.kernel(
        _sc_msg_body,
        mesh=mesh,
        out_type=[
            jax.ShapeDtypeStruct((_N, _H), f32),
            jax.ShapeDtypeStruct((_N, _H), f32),
            jax.ShapeDtypeStruct((_E,), f32),
        ],
        scratch_types=[
            pltpu.VMEM_SHARED((_N, _H), f32),
            pltpu.VMEM((_ZR, _H), f32),
            pltpu.VMEM((_N,), f32),
            pltpu.VMEM((_BE, _H), f32),
            pltpu.VMEM((_BE,), jnp.int32),
            pltpu.VMEM((_BE,), jnp.int32),
            pltpu.VMEM((_BE,), jnp.int32),
            pltpu.VMEM((_BE, _H), f32),
            pltpu.VMEM((_BE,), jnp.int32),
            pltpu.VMEM((_BE,), jnp.int32),
            pltpu.VMEM((_BE,), jnp.int32),
            pltpu.VMEM((_BE,), f32),
            pltpu.VMEM((_BE,), f32),
            pltpu.VMEM((_BE,), f32),
            pltpu.SemaphoreType.DMA,
            pltpu.SemaphoreType.DMA,
            pltpu.SemaphoreType.DMA,
            pltpu.SemaphoreType.DMA,
        ],
        compiler_params=_sc_compiler_params(),
    )
    return kern(val2, cos, ct, ispt, row, col)


def _stage2_body(out0_ref, out1_ref, ideg_ref, res_ref, wo_ref, o_ref):
    out = jnp.concatenate([out0_ref[...], out1_ref[...]], axis=1)
    out = out * ideg_ref[...]
    wo = wo_ref[...]
    lin = jnp.dot(out, wo.T, preferred_element_type=jnp.float32)
    onrm = jnp.sqrt(jnp.sum(out * out, axis=1, keepdims=True))
    out_n = out / jnp.clip(onrm, 1e-12, None)
    wnrm = jnp.sqrt(jnp.sum(wo * wo, axis=1, keepdims=True))
    w_n = wo / jnp.clip(wnrm, 1e-12, None)
    cos2 = jnp.clip(jnp.dot(out_n, w_n.T, preferred_element_type=jnp.float32),
                    _EPS, 1.0)
    o_ref[...] = lin * cos2 + res_ref[...]


def _stage2(out0, out1, ideg, res, Wo):
    return pl.pallas_call(
        _stage2_body,
        grid=(_N // _BLK,),
        in_specs=[
            pl.BlockSpec((_BLK, _H), lambda i: (i, 0)),
            pl.BlockSpec((_BLK, _H), lambda i: (i, 0)),
            pl.BlockSpec((_BLK, 1), lambda i: (i, 0)),
            pl.BlockSpec((_BLK, _C), lambda i: (i, 0)),
            pl.BlockSpec((_C, _C), lambda i: (0, 0)),
        ],
        out_specs=pl.BlockSpec((_BLK, _C), lambda i: (i, 0)),
        out_shape=jax.ShapeDtypeStruct((_N, _C), jnp.float32),
    )(out0, out1, ideg, res, Wo)


def kernel(x, edge_index, Wv, Wo, Wr):
    row = edge_index[0].astype(jnp.int32)
    col = edge_index[1].astype(jnp.int32)
    val, norm, res, s1 = _stage1(x, Wv, Wr)
    s1 = s1.reshape(_N)

    cos, ct, spt_p, deg_p = _sc_cos(norm, row, col, s1)
    ispt2, ideg = _ispt(spt_p, deg_p)
    ispt = ispt2.reshape(_N)

    val2 = jnp.concatenate([val[:, :_H], val[:, _H:]], axis=0)
    out0, out1, contrib_norm = _sc_msg(val2, cos, ct, ispt, row, col)

    out_final = _stage2(out0, out1, ideg.reshape(_N, 1), res, Wo)
    return (out_final, jax.lax.stop_gradient(contrib_norm))
